# baseline (speedup 1.0000x reference)
import functools

import jax
import jax.numpy as jnp
from jax import lax
from jax.experimental import pallas as pl
from jax.experimental.pallas import tpu as pltpu
from jax.experimental.pallas import tpu_sc as plsc

B = 16
K_TOK = 32768
D_STATE = 256
D_TOKEN = 128

K_TC = K_TOK // 2
K_SC = K_TOK - K_TC
TC_BLK = 2048

NC = 2
NS = 16
NW = NC * NS
KW = K_SC // NW
CH = 256
NCHUNK = KW // CH


def _prep_body(starts_ref, state_ref, wq_ref, bq_ref, wk_ref,
               kqz_ref, bnd_ref, seglim_ref):
    q = lax.dot_general(state_ref[...], wq_ref[...], (((1,), (1,)), ((), ())),
                        preferred_element_type=jnp.float32)
    q = q + bq_ref[...]
    kq = lax.dot_general(q, wk_ref[...], (((1,), (1,)), ((), ())),
                         preferred_element_type=jnp.float32)
    kqz_ref[...] = jnp.zeros((24, D_TOKEN), jnp.float32)
    kqz_ref[0:B, :] = kq
    lo = K_TC + lax.broadcasted_iota(jnp.int32, (NW, 32), 0) * KW
    sj = jnp.zeros((NW, 32), jnp.int32)
    for j in range(B + 1):
        col = (lax.broadcasted_iota(jnp.int32, (NW, 32), 1) == j)
        sj = jnp.where(col, starts_ref[j], sj)
    bnd_ref[...] = jnp.clip(sj - lo, 0, KW)
    row = lax.broadcasted_iota(jnp.int32, (2 * B, D_TOKEN), 0)
    sl = jnp.zeros((2 * B, D_TOKEN), jnp.int32)
    for b in range(B):
        sl = jnp.where(row == b, starts_ref[b], sl)
        sl = jnp.where(row == B + b, starts_ref[b + 1], sl)
    seglim_ref[...] = sl


def _tc_score_body(kq_ref, seglim_ref, cand_ref, out_ref):
    i = pl.program_id(0)
    scores = lax.dot_general(kq_ref[0:B, :], cand_ref[...],
                             (((1,), (1,)), ((), ())),
                             preferred_element_type=jnp.float32)
    tok = i * TC_BLK + lax.broadcasted_iota(jnp.int32, (B, TC_BLK), 1)
    s_lo = seglim_ref[0:B, 0:1]
    s_hi = seglim_ref[B:2 * B, 0:1]
    onehot = ((tok >= s_lo) & (tok < s_hi)).astype(jnp.float32)
    out_ref[...] = jnp.sum(scores * onehot, axis=0)


def _sc_body(kq_hbm, bnd_hbm, cand_hbm, out_hbm,
             kq_v, buf0, buf1, bnd_v, out_v, sem0, sem1):
    cid = lax.axis_index("c")
    sid = lax.axis_index("s")
    wid = sid * NC + cid
    base_tok = pl.multiple_of(wid * KW, KW)

    pltpu.sync_copy(kq_hbm, kq_v)
    pltpu.sync_copy(bnd_hbm.at[wid], bnd_v)

    bv0 = bnd_v[pl.ds(0, 16)]
    bv1 = bnd_v[pl.ds(16, 16)]
    A = [bv0[j] for j in range(16)] + [bv1[0]]

    lane = lax.iota(jnp.int32, 16)
    last_lane = lane == 15
    zero16 = jnp.zeros((16,), jnp.float32)

    for z in range(KW // 16):
        out_v[pl.ds(z * 16, 16)] = zero16

    bufs = (buf0, buf1)
    sems = (sem0, sem1)

    def chunk_src(c):
        off = pl.multiple_of(K_TC + base_tok + c * CH, CH)
        return cand_hbm.at[pl.ds(off, CH)]

    desc = [None, None]
    desc[0] = pltpu.async_copy(chunk_src(0), bufs[0], sems[0])

    for c in range(NCHUNK):
        if c + 1 < NCHUNK:
            desc[(c + 1) % 2] = pltpu.async_copy(
                chunk_src(c + 1), bufs[(c + 1) % 2], sems[(c + 1) % 2])
        desc[c % 2].wait()
        buf = bufs[c % 2]

        for j in range(B):
            lo = jnp.maximum(A[j], c * CH)
            hi = jnp.minimum(A[j + 1], (c + 1) * CH)
            hi = jnp.maximum(lo, hi)
            kqc = [kq_v[j, pl.ds(cc * 16, 16)] for cc in range(8)]

            def tok_body(t, buf=buf, c=c, kqc=kqc):
                r = t - c * CH
                acc0 = buf[r, pl.ds(0, 16)] * kqc[0]
                acc1 = buf[r, pl.ds(16, 16)] * kqc[1]
                acc2 = buf[r, pl.ds(32, 16)] * kqc[2]
                acc3 = buf[r, pl.ds(48, 16)] * kqc[3]
                acc0 = acc0 + buf[r, pl.ds(64, 16)] * kqc[4]
                acc1 = acc1 + buf[r, pl.ds(80, 16)] * kqc[5]
                acc2 = acc2 + buf[r, pl.ds(96, 16)] * kqc[6]
                acc3 = acc3 + buf[r, pl.ds(112, 16)] * kqc[7]
                acc = (acc0 + acc1) + (acc2 + acc3)
                s = jnp.cumsum(acc)
                idx = jnp.zeros((16,), jnp.int32) + t
                plsc.store_scatter(out_v, [idx], s, mask=last_lane)

            plsc.parallel_loop(lo, hi, 1, unroll=4)(tok_body)

    pltpu.sync_copy(out_v, out_hbm.at[pl.ds(base_tok, KW)])


@jax.jit
def kernel(state_vec, cand_tokens, starts, Wq, bq, Wk):
    starts_i = starts.astype(jnp.int32)
    kqz, bnd, seglim = pl.pallas_call(
        _prep_body,
        out_shape=[
            jax.ShapeDtypeStruct((24, D_TOKEN), jnp.float32),
            jax.ShapeDtypeStruct((NW, 32), jnp.int32),
            jax.ShapeDtypeStruct((2 * B, D_TOKEN), jnp.int32),
        ],
        in_specs=[
            pl.BlockSpec(memory_space=pltpu.SMEM),
            pl.BlockSpec(memory_space=pltpu.VMEM),
            pl.BlockSpec(memory_space=pltpu.VMEM),
            pl.BlockSpec(memory_space=pltpu.VMEM),
            pl.BlockSpec(memory_space=pltpu.VMEM),
        ],
        out_specs=[
            pl.BlockSpec(memory_space=pltpu.VMEM),
            pl.BlockSpec(memory_space=pltpu.VMEM),
            pl.BlockSpec(memory_space=pltpu.VMEM),
        ],
    )(starts_i, state_vec, Wq, bq.reshape(1, D_TOKEN), Wk)

    mesh = plsc.VectorSubcoreMesh(core_axis_name="c", subcore_axis_name="s",
                                  num_cores=NC, num_subcores=NS)
    sc = pl.kernel(
        _sc_body,
        out_type=jax.ShapeDtypeStruct((K_SC,), jnp.float32),
        mesh=mesh,
        compiler_params=pltpu.CompilerParams(needs_layout_passes=False),
        scratch_types=[
            pltpu.VMEM((24, D_TOKEN), jnp.float32),
            pltpu.VMEM((CH, D_TOKEN), jnp.float32),
            pltpu.VMEM((CH, D_TOKEN), jnp.float32),
            pltpu.VMEM((32,), jnp.int32),
            pltpu.VMEM((KW,), jnp.float32),
            pltpu.SemaphoreType.DMA,
            pltpu.SemaphoreType.DMA,
        ],
    )
    sc_half = sc(kqz, bnd, cand_tokens)

    tc_half = pl.pallas_call(
        _tc_score_body,
        grid=(K_TC // TC_BLK,),
        out_shape=jax.ShapeDtypeStruct((K_TC,), jnp.float32),
        in_specs=[
            pl.BlockSpec((24, D_TOKEN), lambda i: (0, 0)),
            pl.BlockSpec((2 * B, D_TOKEN), lambda i: (0, 0)),
            pl.BlockSpec((TC_BLK, D_TOKEN), lambda i: (i, 0)),
        ],
        out_specs=pl.BlockSpec((TC_BLK,), lambda i: (i,)),
    )(kqz, seglim, cand_tokens)

    return jnp.concatenate([tc_half, sc_half])


# speedup vs baseline: 1.2161x; 1.2161x over previous
import functools

import jax
import jax.numpy as jnp
from jax import lax
from jax.experimental import pallas as pl
from jax.experimental.pallas import tpu as pltpu
from jax.experimental.pallas import tpu_sc as plsc

B = 16
K_TOK = 32768
D_STATE = 256
D_TOKEN = 128

NC = 2
NS = 16
NW = NC * NS
KW = K_TOK // NW
CH = 256
NCHUNK = KW // CH


def _prep_body(starts_ref, state_ref, wq_ref, bq_ref, wk_ref, kqz_ref, bnd_ref):
    q = lax.dot_general(state_ref[...], wq_ref[...], (((1,), (1,)), ((), ())),
                        preferred_element_type=jnp.float32)
    q = q + bq_ref[...]
    kq = lax.dot_general(q, wk_ref[...], (((1,), (1,)), ((), ())),
                         preferred_element_type=jnp.float32)
    kqz_ref[...] = jnp.zeros((24, D_TOKEN), jnp.float32)
    kqz_ref[0:B, :] = kq
    lo = lax.broadcasted_iota(jnp.int32, (NW, 32), 0) * KW
    sj = jnp.zeros((NW, 32), jnp.int32)
    for j in range(B + 1):
        col = (lax.broadcasted_iota(jnp.int32, (NW, 32), 1) == j)
        sj = jnp.where(col, starts_ref[j], sj)
    bnd_ref[...] = jnp.clip(sj - lo, 0, KW)


def _sc_body(kq_hbm, bnd_hbm, cand_hbm, out_hbm,
             kq_v, buf0, buf1, buf2, bnd_v, out_v,
             sem0, sem1, sem2, sem3, sem4, sem5):
    cid = lax.axis_index("c")
    sid = lax.axis_index("s")
    wid = sid * NC + cid
    base_tok = pl.multiple_of(wid * KW, KW)

    pltpu.sync_copy(kq_hbm, kq_v)
    pltpu.sync_copy(bnd_hbm.at[wid], bnd_v)


    lane = lax.iota(jnp.int32, 16)
    last_lane = lane == 15
    zero16 = jnp.zeros((16,), jnp.float32)

    for z in range(KW // 16):
        out_v[pl.ds(z * 16, 16)] = zero16

    bufs = (buf0, buf1, buf2)
    sems = ((sem0, sem1), (sem2, sem3), (sem4, sem5))
    NB = 3
    H = CH // 2

    def start_chunk(c):
        off = pl.multiple_of(base_tok + c * CH, CH)
        b = bufs[c % NB]
        s0, s1 = sems[c % NB]
        d0 = pltpu.async_copy(cand_hbm.at[pl.ds(off, H)], b.at[pl.ds(0, H)], s0)
        d1 = pltpu.async_copy(cand_hbm.at[pl.ds(off + H, H)], b.at[pl.ds(H, H)], s1)
        return (d0, d1)

    desc = [None] * NB
    desc[0] = start_chunk(0)
    desc[1] = start_chunk(1)

    for c in range(NCHUNK):
        if c + 2 < NCHUNK:
            desc[(c + 2) % NB] = start_chunk(c + 2)
        desc[c % NB][0].wait()
        desc[c % NB][1].wait()
        buf = bufs[c % NB]

        def run_body(j, _, buf=buf, c=c):
            jv = jnp.zeros((16,), jnp.int32) + j
            a0 = plsc.load_gather(bnd_v, [jv])[0]
            a1 = plsc.load_gather(bnd_v, [jv + 1])[0]
            lo = jnp.maximum(a0, c * CH)
            hi = jnp.minimum(a1, (c + 1) * CH)
            hi = jnp.maximum(lo, hi)
            kqc = [kq_v[j, pl.ds(cc * 16, 16)] for cc in range(8)]

            def tok_body(t, buf=buf, c=c, kqc=kqc):
                r = t - c * CH
                acc0 = buf[r, pl.ds(0, 16)] * kqc[0]
                acc1 = buf[r, pl.ds(16, 16)] * kqc[1]
                acc2 = buf[r, pl.ds(32, 16)] * kqc[2]
                acc3 = buf[r, pl.ds(48, 16)] * kqc[3]
                acc0 = acc0 + buf[r, pl.ds(64, 16)] * kqc[4]
                acc1 = acc1 + buf[r, pl.ds(80, 16)] * kqc[5]
                acc2 = acc2 + buf[r, pl.ds(96, 16)] * kqc[6]
                acc3 = acc3 + buf[r, pl.ds(112, 16)] * kqc[7]
                acc = (acc0 + acc1) + (acc2 + acc3)
                s = jnp.cumsum(acc)
                idx = jnp.zeros((16,), jnp.int32) + t
                plsc.store_scatter(out_v, [idx], s, mask=last_lane)

            plsc.parallel_loop(lo, hi, 1, unroll=4)(tok_body)
            return 0

        lax.fori_loop(0, B, run_body, 0)

    pltpu.sync_copy(out_v, out_hbm.at[pl.ds(base_tok, KW)])


@jax.jit
def kernel(state_vec, cand_tokens, starts, Wq, bq, Wk):
    starts_i = starts.astype(jnp.int32)
    kqz, bnd = pl.pallas_call(
        _prep_body,
        out_shape=[
            jax.ShapeDtypeStruct((24, D_TOKEN), jnp.float32),
            jax.ShapeDtypeStruct((NW, 32), jnp.int32),
        ],
        in_specs=[
            pl.BlockSpec(memory_space=pltpu.SMEM),
            pl.BlockSpec(memory_space=pltpu.VMEM),
            pl.BlockSpec(memory_space=pltpu.VMEM),
            pl.BlockSpec(memory_space=pltpu.VMEM),
            pl.BlockSpec(memory_space=pltpu.VMEM),
        ],
        out_specs=[
            pl.BlockSpec(memory_space=pltpu.VMEM),
            pl.BlockSpec(memory_space=pltpu.VMEM),
        ],
    )(starts_i, state_vec, Wq, bq.reshape(1, D_TOKEN), Wk)

    mesh = plsc.VectorSubcoreMesh(core_axis_name="c", subcore_axis_name="s",
                                  num_cores=NC, num_subcores=NS)
    sc = pl.kernel(
        _sc_body,
        out_type=jax.ShapeDtypeStruct((K_TOK,), jnp.float32),
        mesh=mesh,
        compiler_params=pltpu.CompilerParams(needs_layout_passes=False),
        scratch_types=[
            pltpu.VMEM((24, D_TOKEN), jnp.float32),
            pltpu.VMEM((CH, D_TOKEN), jnp.float32),
            pltpu.VMEM((CH, D_TOKEN), jnp.float32),
            pltpu.VMEM((CH, D_TOKEN), jnp.float32),
            pltpu.VMEM((32,), jnp.int32),
            pltpu.VMEM((KW,), jnp.float32),
            pltpu.SemaphoreType.DMA,
            pltpu.SemaphoreType.DMA,
            pltpu.SemaphoreType.DMA,
            pltpu.SemaphoreType.DMA,
            pltpu.SemaphoreType.DMA,
            pltpu.SemaphoreType.DMA,
        ],
    )
    logits = sc(kqz, bnd, cand_tokens)
    return logits
